# Initial kernel scaffold; baseline (speedup 1.0000x reference)
import jax
import jax.numpy as jnp
from jax.experimental import pallas as pl


def kernel(x_atom, x_attach, x_motif, edge_index_atom, edge_index_attach, edge_index_motif, inter1_src, inter1_dst, inter2_src, inter2_dst, W_hops, W_inter):
    raise NotImplementedError("write your pallas kernel here")


# speedup vs baseline: 6.9222x; 6.9222x over previous
import functools

import jax
import jax.numpy as jnp
from jax import lax
from jax.experimental import pallas as pl
from jax.experimental.pallas import tpu as pltpu
from jax.experimental.pallas import tpu_sc as plsc

D = 128
NC = 2
NS = 16
NW = NC * NS
CHUNK = 128


def _round_up(x, m):
    return (x + m - 1) // m * m


@functools.lru_cache(maxsize=None)
def _make_seg_sum(n_src, n_dst_pad, n_chunks):
    rows_per_sub = n_dst_pad // NS
    mesh = plsc.VectorSubcoreMesh(core_axis_name="c", subcore_axis_name="s")

    @functools.partial(
        pl.kernel,
        out_type=jax.ShapeDtypeStruct((NC, n_dst_pad, D), jnp.float32),
        mesh=mesh,
        scratch_types=[
            pltpu.VMEM((n_chunks, CHUNK), jnp.int32),
            pltpu.VMEM((n_chunks, CHUNK), jnp.int32),
            pltpu.VMEM((CHUNK, D), jnp.float32),
            pltpu.VMEM_SHARED((n_dst_pad, D), jnp.float32),
            pltpu.SemaphoreType.DMA,
        ],
    )
    def seg_sum(h_hbm, srci_hbm, dsti_hbm, zeros_hbm, out_hbm,
                srci_v, dsti_v, rows_v, acc_sh, sem):
        c = lax.axis_index("c")
        s = lax.axis_index("s")
        wid = s * NC + c
        row0 = s * rows_per_sub
        pltpu.sync_copy(zeros_hbm, acc_sh.at[pl.ds(row0, rows_per_sub), :])
        pltpu.sync_copy(srci_hbm.at[wid], srci_v)
        pltpu.sync_copy(dsti_hbm.at[wid], dsti_v)
        plsc.subcore_barrier()

        def body(j, carry):
            pltpu.async_copy(h_hbm.at[srci_v.at[j]], rows_v, sem).wait()
            pltpu.sync_copy(rows_v, acc_sh.at[dsti_v.at[j]], add=True)
            return carry

        lax.fori_loop(0, n_chunks, body, 0)
        plsc.subcore_barrier()
        pltpu.sync_copy(acc_sh.at[pl.ds(row0, rows_per_sub), :],
                        out_hbm.at[c, pl.ds(row0, rows_per_sub), :])

    return seg_sum


_BN = 1024


@functools.lru_cache(maxsize=None)
def _make_hop_mm(n):
    def body(h_ref, p_ref, w_ref, o_ref):
        x = h_ref[...] + p_ref[0] + p_ref[1]
        o_ref[...] = jnp.maximum(
            jnp.dot(x, w_ref[...], preferred_element_type=jnp.float32), 0.0)

    grid = (pl.cdiv(n, _BN),)
    return pl.pallas_call(
        body,
        grid=grid,
        in_specs=[
            pl.BlockSpec((_BN, D), lambda i: (i, 0)),
            pl.BlockSpec((NC, _BN, D), lambda i: (0, i, 0)),
            pl.BlockSpec((D, D), lambda i: (0, 0)),
        ],
        out_specs=pl.BlockSpec((_BN, D), lambda i: (i, 0)),
        out_shape=jax.ShapeDtypeStruct((n, D), jnp.float32),
    )


@functools.lru_cache(maxsize=None)
def _make_inter_mm(n):
    def body(x_ref, p_ref, w_ref, o_ref):
        m = p_ref[0] + p_ref[1]
        o_ref[...] = x_ref[...] + jnp.maximum(
            jnp.dot(m, w_ref[...], preferred_element_type=jnp.float32), 0.0)

    grid = (pl.cdiv(n, _BN),)
    return pl.pallas_call(
        body,
        grid=grid,
        in_specs=[
            pl.BlockSpec((_BN, D), lambda i: (i, 0)),
            pl.BlockSpec((NC, _BN, D), lambda i: (0, i, 0)),
            pl.BlockSpec((D, D), lambda i: (0, 0)),
        ],
        out_specs=pl.BlockSpec((_BN, D), lambda i: (i, 0)),
        out_shape=jax.ShapeDtypeStruct((n, D), jnp.float32),
    )


def _prep_indices(src, dst, n_src, n_dst):
    e = src.shape[0]
    e_pad = _round_up(e, NW * CHUNK)
    n_pad = e_pad - e
    src = src.astype(jnp.int32)
    dst = dst.astype(jnp.int32)
    if n_pad:
        fill = jnp.arange(n_pad, dtype=jnp.int32)
        src = jnp.concatenate([src, fill % n_src])
        dst = jnp.concatenate([dst, n_dst + (fill % 16)])
    n_chunks = e_pad // (NW * CHUNK)
    return (src.reshape(NW, n_chunks, CHUNK),
            dst.reshape(NW, n_chunks, CHUNK), n_chunks)


def _seg_sum(h, src, dst, n_dst):
    n_src = h.shape[0]
    n_dst_pad = _round_up(n_dst + 16, NS * 8)
    srci, dsti, n_chunks = _prep_indices(src, dst, n_src, n_dst)
    zeros = jnp.zeros((n_dst_pad // NS, D), jnp.float32)
    return _make_seg_sum(n_src, n_dst_pad, n_chunks)(h, srci, dsti, zeros)


def _mpn_level(x, edge_index, w_hops_level, n_nodes):
    h = x
    src = edge_index[0]
    dst = edge_index[1]
    mm = _make_hop_mm(n_nodes)
    for hop in range(3):
        partials = _seg_sum(h, src, dst, n_nodes)
        h = mm(h, partials, w_hops_level[hop])
    return h


def kernel(x_atom, x_attach, x_motif, edge_index_atom, edge_index_attach,
           edge_index_motif, inter1_src, inter1_dst, inter2_src, inter2_dst,
           W_hops, W_inter):
    n0, n1, n2 = x_atom.shape[0], x_attach.shape[0], x_motif.shape[0]
    h_atom = _mpn_level(x_atom, edge_index_atom, W_hops[0], n0)

    p1 = _seg_sum(h_atom, inter1_src, inter1_dst, n1)
    x1 = _make_inter_mm(n1)(x_attach, p1, W_inter[0])
    h_attach = _mpn_level(x1, edge_index_attach, W_hops[1], n1)

    p2 = _seg_sum(h_attach, inter2_src, inter2_dst, n2)
    x2 = _make_inter_mm(n2)(x_motif, p2, W_inter[1])
    h_motif = _mpn_level(x2, edge_index_motif, W_hops[2], n2)

    return (h_atom, h_attach, h_motif)


# speedup vs baseline: 8.7694x; 1.2668x over previous
import functools

import jax
import jax.numpy as jnp
from jax import lax
from jax.experimental import pallas as pl
from jax.experimental.pallas import tpu as pltpu
from jax.experimental.pallas import tpu_sc as plsc

D = 128
NC = 2
NS = 16
NW = NC * NS


def _round_up(x, m):
    return (x + m - 1) // m * m


@functools.lru_cache(maxsize=None)
def _make_seg_sum(n_src, n_dst_pad, n_chunks, chunk):
    rows_per_sub = n_dst_pad // NS
    mesh = plsc.VectorSubcoreMesh(core_axis_name="c", subcore_axis_name="s")

    @functools.partial(
        pl.kernel,
        out_type=jax.ShapeDtypeStruct((NC, n_dst_pad, D), jnp.float32),
        mesh=mesh,
        scratch_types=[
            pltpu.VMEM((2, chunk), jnp.int32),
            pltpu.VMEM((2, chunk), jnp.int32),
            pltpu.VMEM((chunk, D), jnp.float32),
            pltpu.VMEM((chunk, D), jnp.float32),
            pltpu.VMEM_SHARED((n_dst_pad, D), jnp.float32),
            pltpu.SemaphoreType.DMA,
            pltpu.SemaphoreType.DMA,
            pltpu.SemaphoreType.DMA,
            pltpu.SemaphoreType.DMA,
        ],
    )
    def seg_sum(h_hbm, idx_hbm, zeros_hbm, out_hbm,
                i0_v, i1_v, r0_v, r1_v, acc_sh, si0, si1, sg0, sg1):
        c = lax.axis_index("c")
        s = lax.axis_index("s")
        wid = s * NC + c
        row0 = s * rows_per_sub
        last = n_chunks - 1

        def idx_cp(j, buf, sem):
            return pltpu.make_async_copy(idx_hbm.at[wid, j], buf, sem)

        def gather(ibuf, rbuf, sem):
            return pltpu.make_async_copy(h_hbm.at[ibuf.at[0]], rbuf, sem)

        def scat(rbuf, ibuf):
            pltpu.sync_copy(rbuf, acc_sh.at[ibuf.at[1]], add=True)

        idx_cp(0, i0_v, si0).start()
        pltpu.sync_copy(zeros_hbm, acc_sh.at[pl.ds(row0, rows_per_sub), :])
        idx_cp(0, i0_v, si0).wait()
        gather(i0_v, r0_v, sg0).start()
        idx_cp(1, i1_v, si1).start()
        plsc.subcore_barrier()

        def body(i, carry):
            j0 = 2 * i
            gather(i0_v, r0_v, sg0).wait()
            idx_cp(j0 + 1, i1_v, si1).wait()
            gather(i1_v, r1_v, sg1).start()
            scat(r0_v, i0_v)
            idx_cp(jnp.minimum(j0 + 2, last), i0_v, si0).start()
            gather(i1_v, r1_v, sg1).wait()
            idx_cp(jnp.minimum(j0 + 2, last), i0_v, si0).wait()
            gather(i0_v, r0_v, sg0).start()
            scat(r1_v, i1_v)
            idx_cp(jnp.minimum(j0 + 3, last), i1_v, si1).start()
            return carry

        lax.fori_loop(0, n_chunks // 2, body, 0)
        gather(i0_v, r0_v, sg0).wait()
        idx_cp(last, i1_v, si1).wait()
        plsc.subcore_barrier()
        pltpu.sync_copy(acc_sh.at[pl.ds(row0, rows_per_sub), :],
                        out_hbm.at[c, pl.ds(row0, rows_per_sub), :])

    return seg_sum


_BN = 1024


@functools.lru_cache(maxsize=None)
def _make_hop_mm(n):
    def body(h_ref, p_ref, w_ref, o_ref):
        x = h_ref[...] + p_ref[0] + p_ref[1]
        o_ref[...] = jnp.maximum(
            jnp.dot(x, w_ref[...], preferred_element_type=jnp.float32), 0.0)

    grid = (pl.cdiv(n, _BN),)
    return pl.pallas_call(
        body,
        grid=grid,
        in_specs=[
            pl.BlockSpec((_BN, D), lambda i: (i, 0)),
            pl.BlockSpec((NC, _BN, D), lambda i: (0, i, 0)),
            pl.BlockSpec((D, D), lambda i: (0, 0)),
        ],
        out_specs=pl.BlockSpec((_BN, D), lambda i: (i, 0)),
        out_shape=jax.ShapeDtypeStruct((n, D), jnp.float32),
    )


@functools.lru_cache(maxsize=None)
def _make_inter_mm(n):
    def body(x_ref, p_ref, w_ref, o_ref):
        m = p_ref[0] + p_ref[1]
        o_ref[...] = x_ref[...] + jnp.maximum(
            jnp.dot(m, w_ref[...], preferred_element_type=jnp.float32), 0.0)

    grid = (pl.cdiv(n, _BN),)
    return pl.pallas_call(
        body,
        grid=grid,
        in_specs=[
            pl.BlockSpec((_BN, D), lambda i: (i, 0)),
            pl.BlockSpec((NC, _BN, D), lambda i: (0, i, 0)),
            pl.BlockSpec((D, D), lambda i: (0, 0)),
        ],
        out_specs=pl.BlockSpec((_BN, D), lambda i: (i, 0)),
        out_shape=jax.ShapeDtypeStruct((n, D), jnp.float32),
    )


CHUNK = 128


def _prep_indices(src, dst, n_src, n_dst):
    e = src.shape[0]
    e_pad = _round_up(e, NW * CHUNK * 2)
    n_pad = e_pad - e
    src = src.astype(jnp.int32)
    dst = dst.astype(jnp.int32)
    if n_pad:
        fill = jnp.arange(n_pad, dtype=jnp.int32)
        src = jnp.concatenate([src, fill % n_src])
        dst = jnp.concatenate([dst, n_dst + (fill % 16)])
    n_chunks = e_pad // (NW * CHUNK)
    idx = jnp.stack([src.reshape(NW, n_chunks, CHUNK),
                     dst.reshape(NW, n_chunks, CHUNK)], axis=2)
    return idx, n_chunks


def _seg_sum(h, src, dst, n_dst):
    n_src = h.shape[0]
    n_dst_pad = _round_up(n_dst + 16, NS * 8)
    idx, n_chunks = _prep_indices(src, dst, n_src, n_dst)
    zeros = jnp.zeros((n_dst_pad // NS, D), jnp.float32)
    return _make_seg_sum(n_src, n_dst_pad, n_chunks, CHUNK)(h, idx, zeros)


def _mpn_level(x, edge_index, w_hops_level, n_nodes):
    h = x
    src = edge_index[0]
    dst = edge_index[1]
    mm = _make_hop_mm(n_nodes)
    for hop in range(3):
        partials = _seg_sum(h, src, dst, n_nodes)
        h = mm(h, partials, w_hops_level[hop])
    return h


def kernel(x_atom, x_attach, x_motif, edge_index_atom, edge_index_attach,
           edge_index_motif, inter1_src, inter1_dst, inter2_src, inter2_dst,
           W_hops, W_inter):
    n0, n1, n2 = x_atom.shape[0], x_attach.shape[0], x_motif.shape[0]
    h_atom = _mpn_level(x_atom, edge_index_atom, W_hops[0], n0)

    p1 = _seg_sum(h_atom, inter1_src, inter1_dst, n1)
    x1 = _make_inter_mm(n1)(x_attach, p1, W_inter[0])
    h_attach = _mpn_level(x1, edge_index_attach, W_hops[1], n1)

    p2 = _seg_sum(h_attach, inter2_src, inter2_dst, n2)
    x2 = _make_inter_mm(n2)(x_motif, p2, W_inter[1])
    h_motif = _mpn_level(x2, edge_index_motif, W_hops[2], n2)

    return (h_atom, h_attach, h_motif)
